# baseline (speedup 1.0000x reference)
import functools

import jax
import jax.numpy as jnp
from jax.experimental import pallas as pl
from jax.experimental.pallas import tpu as pltpu

_UPDATE_RATE = 0.5
_AGE_FACTOR = 0.98

_DN_T = (((1,), (1,)), ((), ()))


def _dot_t(a, b):
    return jax.lax.dot_general(a, b, _DN_T, preferred_element_type=jnp.float32)


def _body(S, B, NS, M,
          hs_ref, mem0_ref,
          win_ref, wval_ref, wg_ref, wu_ref, wr_ref,
          bin_ref, bg_ref, bu_ref, br_ref,
          out_ref,
          min_scr, xg_scr, xu_scr, xr_scr):
    f32 = jnp.float32

    hs = hs_ref[...].reshape(B * S, -1)
    m_in_all = _dot_t(hs, win_ref[...]) + bin_ref[...]
    vals = _dot_t(hs, wval_ref[...])
    xg_all = _dot_t(vals, wg_ref[:, :M]) + bg_ref[...]
    xu_all = _dot_t(vals, wu_ref[:, :M]) + bu_ref[...]
    xr_all = _dot_t(vals, wr_ref[:, :M]) + br_ref[...]
    for b in range(B):
        sl = slice(b * S, (b + 1) * S)
        min_scr[:S, b, :] = m_in_all[sl]
        xg_scr[:, b, :] = xg_all[sl]
        xu_scr[:, b, :] = xu_all[sl]
        xr_scr[:, b, :] = xr_all[sl]
    min_scr[S:] = jnp.zeros((1, B, M), dtype=f32)

    wgh = wg_ref[:, M:]
    wuh = wu_ref[:, M:]
    wrh = wr_ref[:, M:]

    H = B // 2

    def gru_chunk(UC, invC, xrC, xgC, xuC, ww3C, m_in_nextC):
        U2 = UC.reshape(H * NS, M)
        r_pre = _dot_t(U2, wrh).reshape(H, NS, M)
        g_pre = _dot_t(U2, wgh).reshape(H, NS, M)
        reset = jax.nn.sigmoid(invC * r_pre + xrC[:, None, :])
        upd = jax.nn.sigmoid(invC * g_pre + xgC[:, None, :])
        memn = UC * invC
        rh = (reset * memn).reshape(H * NS, M)
        cand = jnp.tanh(_dot_t(rh, wuh).reshape(H, NS, M) + xuC[:, None, :])
        Unew = memn + (ww3C * upd) * (cand - memn)
        nsq = jnp.sum(Unew * Unew, axis=2, keepdims=True)
        invn = jax.lax.rsqrt(jnp.maximum(nsq, 1e-24))
        dotn = jnp.sum(Unew * m_in_nextC[:, None, :], axis=2, keepdims=True)
        sim_next = (dotn * invn)[:, :, 0]
        return Unew, invn, sim_next

    def step(t, carry):
        UA, UB, invA, invB, simA, simB, usage, age = carry
        xg = xg_scr[t]
        xu = xu_scr[t]
        xr = xr_scr[t]
        m_in_next = min_scr[t + 1]

        sim = jnp.concatenate([simA, simB], axis=0)
        scores = usage * 0.2 + age * 0.1 - sim
        w = scores - jnp.max(scores, axis=1, keepdims=True)
        e = jnp.exp(w)
        write_w = e / jnp.sum(e, axis=1, keepdims=True)
        wwm = jnp.where(write_w > 0.01, write_w, jnp.zeros_like(write_w))
        ww3 = (wwm * _UPDATE_RATE)[:, :, None]

        UnA, invnA, simnA = gru_chunk(UA, invA, xr[:H], xg[:H], xu[:H],
                                      ww3[:H], m_in_next[:H])
        UnB, invnB, simnB = gru_chunk(UB, invB, xr[H:], xg[H:], xu[H:],
                                      ww3[H:], m_in_next[H:])

        usage = (usage + wwm) * 0.99
        age = age * _AGE_FACTOR + 1.0
        return UnA, UnB, invnA, invnB, simnA, simnB, usage, age

    zeros = jnp.zeros((B, NS), dtype=f32)
    mem0A = mem0_ref[:H]
    mem0B = mem0_ref[H:]
    m_in0 = min_scr[0]
    sim0A = jnp.sum(mem0A * m_in0[:H, None, :], axis=2)
    sim0B = jnp.sum(mem0B * m_in0[H:, None, :], axis=2)
    ones = jnp.ones((H, NS, 1), dtype=f32)
    UA, UB, invA, invB, _, _, _, _ = jax.lax.fori_loop(
        0, S, step, (mem0A, mem0B, ones, ones, sim0A, sim0B, zeros, zeros),
        unroll=4)
    out_ref[:H] = UA * invA
    out_ref[H:] = UB * invB


@jax.jit
def kernel(hidden_states, memory0, W_in, b_in, W_val, b_val,
           W_gate, b_gate, W_upd, b_upd, W_reset, b_reset):
    B, S, D = hidden_states.shape
    _, NS, M = memory0.shape

    bg = (b_gate + W_gate[:, :M] @ b_val).reshape(1, M)
    bu = (b_upd + W_upd[:, :M] @ b_val).reshape(1, M)
    br = (b_reset + W_reset[:, :M] @ b_val).reshape(1, M)

    body = functools.partial(_body, S, B, NS, M)
    out = pl.pallas_call(
        body,
        out_shape=jax.ShapeDtypeStruct((B, NS, M), jnp.float32),
        scratch_shapes=[pltpu.VMEM((S + 1, B, M), jnp.float32),
                        pltpu.VMEM((S, B, M), jnp.float32),
                        pltpu.VMEM((S, B, M), jnp.float32),
                        pltpu.VMEM((S, B, M), jnp.float32)],
    )(hidden_states, memory0,
      W_in, W_val, W_gate, W_upd, W_reset,
      b_in.reshape(1, M), bg, bu, br)
    return out


# speedup vs baseline: 1.4096x; 1.4096x over previous
import functools

import jax
import jax.numpy as jnp
from jax.experimental import pallas as pl
from jax.experimental.pallas import tpu as pltpu

_UPDATE_RATE = 0.5
_AGE_FACTOR = 0.98

_DN_T = (((1,), (1,)), ((), ()))


def _dot_t(a, b):
    return jax.lax.dot_general(a, b, _DN_T, preferred_element_type=jnp.float32)


def _body(S, B, NS, M,
          hs_ref, mem0_ref,
          win_ref, wval_ref, wg_ref, wu_ref, wr_ref,
          bin_ref, bg_ref, bu_ref, br_ref,
          out_ref,
          min_scr, xg_scr, xu_scr, xr_scr,
          wgh_scr, wuh_scr, wrh_scr):
    f32 = jnp.float32

    hs = hs_ref[...].reshape(B * S, -1)
    m_in_all = _dot_t(hs, win_ref[...]) + bin_ref[...]
    vals = _dot_t(hs, wval_ref[...])
    xg_all = _dot_t(vals, wg_ref[:, :M]) + bg_ref[...]
    xu_all = _dot_t(vals, wu_ref[:, :M]) + bu_ref[...]
    xr_all = _dot_t(vals, wr_ref[:, :M]) + br_ref[...]
    for b in range(B):
        sl = slice(b * S, (b + 1) * S)
        min_scr[:S, b, :] = m_in_all[sl]
        xg_scr[:, b, :] = xg_all[sl]
        xu_scr[:, b, :] = xu_all[sl]
        xr_scr[:, b, :] = xr_all[sl]
    min_scr[S:] = jnp.zeros((1, B, M), dtype=f32)

    wgh_scr[...] = wg_ref[:, M:].T
    wuh_scr[...] = wu_ref[:, M:].T
    wrh_scr[...] = wr_ref[:, M:].T
    wgh = wgh_scr[...]
    wuh = wuh_scr[...]
    wrh = wrh_scr[...]

    H = B // 2

    def gru_chunk(UC, invC, xrC, xgC, xuC, ww3C, m_in_nextC):
        U2 = UC.reshape(H * NS, M)
        r_pre = jnp.dot(U2, wrh, preferred_element_type=f32).reshape(H, NS, M)
        g_pre = jnp.dot(U2, wgh, preferred_element_type=f32).reshape(H, NS, M)
        reset = jax.nn.sigmoid(invC * r_pre + xrC[:, None, :])
        upd = jax.nn.sigmoid(invC * g_pre + xgC[:, None, :])
        memn = UC * invC
        rh = (reset * memn).reshape(H * NS, M)
        cand = jnp.tanh(
            jnp.dot(rh, wuh, preferred_element_type=f32).reshape(H, NS, M)
            + xuC[:, None, :])
        Unew = memn + (ww3C * upd) * (cand - memn)
        nsq = jnp.sum(Unew * Unew, axis=2, keepdims=True)
        invn = jax.lax.rsqrt(jnp.maximum(nsq, 1e-24))
        dotn = jnp.sum(Unew * m_in_nextC[:, None, :], axis=2, keepdims=True)
        sim_next = (dotn * invn)[:, :, 0]
        return Unew, invn, sim_next

    def step(t, carry):
        UA, UB, invA, invB, simA, simB, usage, age = carry
        xg = xg_scr[t]
        xu = xu_scr[t]
        xr = xr_scr[t]
        m_in_next = min_scr[t + 1]

        sim = jnp.concatenate([simA, simB], axis=0)
        scores = usage * 0.2 + age * 0.1 - sim
        w = scores - jnp.max(scores, axis=1, keepdims=True)
        e = jnp.exp(w)
        write_w = e / jnp.sum(e, axis=1, keepdims=True)
        wwm = jnp.where(write_w > 0.01, write_w, jnp.zeros_like(write_w))
        ww3 = (wwm * _UPDATE_RATE)[:, :, None]

        UnA, invnA, simnA = gru_chunk(UA, invA, xr[:H], xg[:H], xu[:H],
                                      ww3[:H], m_in_next[:H])
        UnB, invnB, simnB = gru_chunk(UB, invB, xr[H:], xg[H:], xu[H:],
                                      ww3[H:], m_in_next[H:])

        usage = (usage + wwm) * 0.99
        age = age * _AGE_FACTOR + 1.0
        return UnA, UnB, invnA, invnB, simnA, simnB, usage, age

    zeros = jnp.zeros((B, NS), dtype=f32)
    mem0A = mem0_ref[:H]
    mem0B = mem0_ref[H:]
    m_in0 = min_scr[0]
    sim0A = jnp.sum(mem0A * m_in0[:H, None, :], axis=2)
    sim0B = jnp.sum(mem0B * m_in0[H:, None, :], axis=2)
    ones = jnp.ones((H, NS, 1), dtype=f32)
    UA, UB, invA, invB, _, _, _, _ = jax.lax.fori_loop(
        0, S, step, (mem0A, mem0B, ones, ones, sim0A, sim0B, zeros, zeros),
        unroll=4)
    out_ref[:H] = UA * invA
    out_ref[H:] = UB * invB


@jax.jit
def kernel(hidden_states, memory0, W_in, b_in, W_val, b_val,
           W_gate, b_gate, W_upd, b_upd, W_reset, b_reset):
    B, S, D = hidden_states.shape
    _, NS, M = memory0.shape

    bg = (b_gate + W_gate[:, :M] @ b_val).reshape(1, M)
    bu = (b_upd + W_upd[:, :M] @ b_val).reshape(1, M)
    br = (b_reset + W_reset[:, :M] @ b_val).reshape(1, M)

    body = functools.partial(_body, S, B, NS, M)
    out = pl.pallas_call(
        body,
        out_shape=jax.ShapeDtypeStruct((B, NS, M), jnp.float32),
        scratch_shapes=[pltpu.VMEM((S + 1, B, M), jnp.float32),
                        pltpu.VMEM((S, B, M), jnp.float32),
                        pltpu.VMEM((S, B, M), jnp.float32),
                        pltpu.VMEM((S, B, M), jnp.float32),
                        pltpu.VMEM((M, M), jnp.float32),
                        pltpu.VMEM((M, M), jnp.float32),
                        pltpu.VMEM((M, M), jnp.float32)],
    )(hidden_states, memory0,
      W_in, W_val, W_gate, W_upd, W_reset,
      b_in.reshape(1, M), bg, bu, br)
    return out


# speedup vs baseline: 1.6590x; 1.1769x over previous
import functools

import jax
import jax.numpy as jnp
from jax.experimental import pallas as pl
from jax.experimental.pallas import tpu as pltpu

_UPDATE_RATE = 0.5
_AGE_FACTOR = 0.98

_DN_T = (((1,), (1,)), ((), ()))


def _dot_t(a, b):
    return jax.lax.dot_general(a, b, _DN_T, preferred_element_type=jnp.float32)


def _body(S, B, NS, M,
          hs_ref, mem0_ref,
          win_ref, wval_ref, wg_ref, wu_ref, wr_ref,
          bin_ref, bval_ref, bg_ref, bu_ref, br_ref,
          out_ref,
          min_scr, xg_scr, xu_scr, xr_scr,
          wgh_scr, wuh_scr, wrh_scr):
    f32 = jnp.float32

    hs = hs_ref[...].reshape(B * S, -1)
    m_in_all = _dot_t(hs, win_ref[...]) + bin_ref[...]
    vals = _dot_t(hs, wval_ref[...]) + bval_ref[...]
    xg_all = _dot_t(vals, wg_ref[:, :M]) + bg_ref[...]
    xu_all = _dot_t(vals, wu_ref[:, :M]) + bu_ref[...]
    xr_all = _dot_t(vals, wr_ref[:, :M]) + br_ref[...]
    for b in range(B):
        sl = slice(b * S, (b + 1) * S)
        min_scr[:S, b, :] = m_in_all[sl]
        xg_scr[:, b, :] = xg_all[sl]
        xu_scr[:, b, :] = xu_all[sl]
        xr_scr[:, b, :] = xr_all[sl]
    min_scr[S:] = jnp.zeros((1, B, M), dtype=f32)

    wgh_scr[...] = wg_ref[:, M:].T
    wuh_scr[...] = wu_ref[:, M:].T
    wrh_scr[...] = wr_ref[:, M:].T
    wgh = wgh_scr[...]
    wuh = wuh_scr[...]
    wrh = wrh_scr[...]

    H = B // 2

    def gru_chunk(UC, invC, xrC, xgC, xuC, ww3C, m_in_nextC):
        U2 = UC.reshape(H * NS, M)
        r_pre = jnp.dot(U2, wrh, preferred_element_type=f32).reshape(H, NS, M)
        g_pre = jnp.dot(U2, wgh, preferred_element_type=f32).reshape(H, NS, M)
        reset = jax.nn.sigmoid(invC * r_pre + xrC[:, None, :])
        upd = jax.nn.sigmoid(invC * g_pre + xgC[:, None, :])
        memn = UC * invC
        rh = (reset * memn).reshape(H * NS, M)
        cand = jnp.tanh(
            jnp.dot(rh, wuh, preferred_element_type=f32).reshape(H, NS, M)
            + xuC[:, None, :])
        Unew = memn + (ww3C * upd) * (cand - memn)
        nsq = jnp.sum(Unew * Unew, axis=2, keepdims=True)
        invn = jax.lax.rsqrt(jnp.maximum(nsq, 1e-24))
        dotn = jnp.sum(Unew * m_in_nextC[:, None, :], axis=2, keepdims=True)
        sim_next = (dotn * invn)[:, :, 0]
        return Unew, invn, sim_next

    def step(t, carry):
        UA, UB, invA, invB, simA, simB, usage, age = carry
        xg = xg_scr[t]
        xu = xu_scr[t]
        xr = xr_scr[t]
        m_in_next = min_scr[t + 1]

        sim = jnp.concatenate([simA, simB], axis=0)
        scores = usage * 0.2 + age * 0.1 - sim
        w = scores - jnp.max(scores, axis=1, keepdims=True)
        e = jnp.exp(w)
        write_w = e / jnp.sum(e, axis=1, keepdims=True)
        wwm = jnp.where(write_w > 0.01, write_w, jnp.zeros_like(write_w))
        ww3 = (wwm * _UPDATE_RATE)[:, :, None]

        UnA, invnA, simnA = gru_chunk(UA, invA, xr[:H], xg[:H], xu[:H],
                                      ww3[:H], m_in_next[:H])
        UnB, invnB, simnB = gru_chunk(UB, invB, xr[H:], xg[H:], xu[H:],
                                      ww3[H:], m_in_next[H:])

        usage = (usage + wwm) * 0.99
        age = age * _AGE_FACTOR + 1.0
        return UnA, UnB, invnA, invnB, simnA, simnB, usage, age

    zeros = jnp.zeros((B, NS), dtype=f32)
    mem0A = mem0_ref[:H]
    mem0B = mem0_ref[H:]
    m_in0 = min_scr[0]
    sim0A = jnp.sum(mem0A * m_in0[:H, None, :], axis=2)
    sim0B = jnp.sum(mem0B * m_in0[H:, None, :], axis=2)
    ones = jnp.ones((H, NS, 1), dtype=f32)
    UA, UB, invA, invB, _, _, _, _ = jax.lax.fori_loop(
        0, S, step, (mem0A, mem0B, ones, ones, sim0A, sim0B, zeros, zeros),
        unroll=4)
    out_ref[:H] = UA * invA
    out_ref[H:] = UB * invB


@jax.jit
def kernel(hidden_states, memory0, W_in, b_in, W_val, b_val,
           W_gate, b_gate, W_upd, b_upd, W_reset, b_reset):
    B, S, D = hidden_states.shape
    _, NS, M = memory0.shape

    body = functools.partial(_body, S, B, NS, M)
    out = pl.pallas_call(
        body,
        out_shape=jax.ShapeDtypeStruct((B, NS, M), jnp.float32),
        scratch_shapes=[pltpu.VMEM((S + 1, B, M), jnp.float32),
                        pltpu.VMEM((S, B, M), jnp.float32),
                        pltpu.VMEM((S, B, M), jnp.float32),
                        pltpu.VMEM((S, B, M), jnp.float32),
                        pltpu.VMEM((M, M), jnp.float32),
                        pltpu.VMEM((M, M), jnp.float32),
                        pltpu.VMEM((M, M), jnp.float32)],
    )(hidden_states, memory0,
      W_in, W_val, W_gate, W_upd, W_reset,
      b_in.reshape(1, M), b_val.reshape(1, M), b_gate.reshape(1, M),
      b_upd.reshape(1, M), b_reset.reshape(1, M))
    return out
